# baseline (speedup 1.0000x reference)
import functools

import jax
import jax.numpy as jnp
from jax import lax
from jax.experimental import pallas as pl
from jax.experimental.pallas import tpu as pltpu
from jax.experimental.pallas import tpu_sc as plsc

B = 16384
NEG = -1e30
NC = 2
NS = 16
NW = NC * NS
BH = B // 2
RPW = BH // NW
CH = 64
NCH = RPW // CH

NSEC = 6
SECROWS = NSEC * RPW

POUT = 128
F_T, F_A, F_V, F_S = 0, 7, 14, 21
I_T, I_A, I_V, I_S = 24, 31, 38, 45
W_T, W_A, W_V, W_S = 47, 54, 61, 68


def _scores_body(tT, aT, vT, sT,
                 tW1, tb1, tW2, tb2,
                 aW1, ab1, aW2, ab2,
                 vW1, vb1, vW2, vb2,
                 sW1, sb1, sW2, sb2,
                 o):
    def mlp_t(xT, W1t, b1c, W2t, b2c):
        h = jnp.maximum(
            jnp.dot(W1t[...], xT, preferred_element_type=jnp.float32)
            + b1c[...], 0.0)
        return jnp.dot(W2t[...], h, preferred_element_type=jnp.float32) + b2c[...]

    tv, av, vv, sv = tT[...], aT[...], vT[...], sT[...]
    R = RPW
    o[0:R, :] = jnp.transpose(mlp_t(tv, tW1, tb1, tW2, tb2))
    o[R:2 * R, :] = jnp.transpose(mlp_t(av, aW1, ab1, aW2, ab2))
    o[2 * R:3 * R, :] = jnp.transpose(mlp_t(vv, vW1, vb1, vW2, vb2))
    o[3 * R:4 * R, 0:16] = jnp.transpose(mlp_t(sv, sW1, sb1, sW2, sb2))
    o[3 * R:4 * R, 16:84] = jnp.transpose(tv)
    o[3 * R:4 * R, 84:100] = jnp.transpose(sv)
    o[4 * R:5 * R, 0:68] = jnp.transpose(av)
    o[5 * R:6 * R, 0:68] = jnp.transpose(vv)


def _tc_scores(tT, aT, vT, sT, tw, aw, vw, sw, h):
    grid = (NW,)

    def dataT_spec(rows):
        return pl.BlockSpec((rows, RPW), lambda i: (0, i + h * NW))

    def full_spec(arr):
        return pl.BlockSpec(arr.shape, lambda i: (0,) * arr.ndim)

    in_specs = [dataT_spec(68), dataT_spec(68), dataT_spec(68), dataT_spec(16)]
    ws = list(tw) + list(aw) + list(vw) + list(sw)
    in_specs += [full_spec(w) for w in ws]
    return pl.pallas_call(
        _scores_body, grid=grid, in_specs=in_specs,
        out_specs=pl.BlockSpec((SECROWS, POUT), lambda i: (i, 0)),
        out_shape=jax.ShapeDtypeStruct((NW * SECROWS, POUT), jnp.float32),
    )(tT, aT, vT, sT, *ws)


def _merge(ka, va, kb, vb):
    rkb = jnp.flip(kb)
    rvb = jnp.flip(vb)
    c = ka >= rkb
    hk = jnp.where(c, ka, rkb)
    hv = jnp.where(c, va, rvb)
    return plsc.sort_key_val(hk, hv, descending=True)


def _topk_row(sc_ref, r, sc_off, nchunk, lane):
    ks, vs = [], []
    for j in range(nchunk):
        key = sc_ref[r, pl.ds(sc_off + j * 16, 16)]
        kk, vv = plsc.sort_key_val(key, lane + j * 16, descending=True)
        ks.append(kk)
        vs.append(vv)
    while len(ks) > 1:
        nk, nv = [], []
        for i in range(0, len(ks) - 1, 2):
            kk, vv = _merge(ks[i], vs[i], ks[i + 1], vs[i + 1])
            nk.append(kk)
            nv.append(vv)
        if len(ks) % 2:
            nk.append(ks[-1])
            nv.append(vs[-1])
        ks, vs = nk, nv
    return ks[0], vs[0]


def _branch_row(sc_ref, sc_off, x_ref, x_off, po, r, rvec, nchunk, mk, lane,
                f_off, i_off, w_off, fm):
    keys, vals = _topk_row(sc_ref, r, sc_off, nchunk, lane)
    mx = jnp.max(keys)
    e = jnp.where(mk, jnp.exp(keys - mx), 0.0)
    w = e / jnp.sum(e)
    idx = jnp.where(mk, vals, 0)
    xs = plsc.load_gather(x_ref, [rvec, x_off + idx], mask=mk)
    wt = jnp.where(mk, xs * w, 0.0)
    plsc.store_scatter(po, [rvec, lane + i_off],
                       plsc.bitcast(idx, jnp.float32), mask=mk)
    plsc.store_scatter(po, [rvec, lane + w_off], w, mask=mk)
    plsc.store_scatter(po, [rvec, lane + f_off], wt, mask=fm)


def _sc_body(pin_h, pout_h,
             ct0, ca0, cv0, xd0, xe0, xf0,
             ct1, ca1, cv1, xd1, xe1, xf1,
             po0, po1, si0, si1, so0, so1):
    wid = lax.axis_index("s") * NC + lax.axis_index("c")
    lane = lax.iota(jnp.int32, 16)
    m7 = lane < 7
    m2 = lane < 2
    m3 = lane < 3
    secbase = wid * SECROWS

    bufs = ((ct0, ca0, cv0, xd0, xe0, xf0),
            (ct1, ca1, cv1, xd1, xe1, xf1))
    pos = (po0, po1)
    sis = (si0, si1)
    sos = (so0, so1)

    def start_in(c, bsel):
        sem = sis[bsel]
        for s, dst in enumerate(bufs[bsel]):
            pltpu.async_copy(
                pin_h.at[pl.ds(secbase + s * RPW + c * CH, CH)], dst, sem)

    def wait_in(bsel):
        for dst in bufs[bsel]:
            pltpu.make_async_copy(pin_h.at[pl.ds(0, CH)], dst,
                                  sis[bsel]).wait()

    def start_out(c, bsel):
        pltpu.async_copy(
            pos[bsel], pout_h.at[pl.ds(wid * RPW + c * CH, CH)], sos[bsel])

    def wait_out(bsel):
        pltpu.make_async_copy(pout_h.at[pl.ds(0, CH)], pos[bsel],
                              sos[bsel]).wait()

    def compute_chunk(bsel):
        sct, sca, scv, xd, xe, xf = bufs[bsel]
        po = pos[bsel]

        @plsc.parallel_loop(0, CH, unroll=2)
        def _(r):
            rvec = jnp.full((16,), r, jnp.int32)
            _branch_row(sct, 0, xd, 16, po, r, rvec, 5, m7, lane,
                        F_T, I_T, W_T, m7)
            _branch_row(sca, 0, xe, 0, po, r, rvec, 5, m7, lane,
                        F_A, I_A, W_A, m7)
            _branch_row(scv, 0, xf, 0, po, r, rvec, 5, m7, lane,
                        F_V, I_V, W_V, m7)
            _branch_row(xd, 0, xd, 84, po, r, rvec, 1, m2, lane,
                        F_S, I_S, W_S, m3)

    start_in(0, 0)

    def pair_body(c2, _):
        c = 2 * c2
        start_in(c + 1, 1)
        wait_in(0)

        @pl.when(c2 > 0)
        def _():
            wait_out(0)

        compute_chunk(0)
        start_out(c, 0)

        @pl.when(c2 < NCH // 2 - 1)
        def _():
            start_in(c + 2, 0)

        wait_in(1)

        @pl.when(c2 > 0)
        def _():
            wait_out(1)

        compute_chunk(1)
        start_out(c + 1, 1)
        return 0

    lax.fori_loop(0, NCH // 2, pair_body, 0)
    wait_out(0)
    wait_out(1)


def _sc_topk(pin):
    f32 = jnp.float32
    mesh = plsc.VectorSubcoreMesh(core_axis_name="c", subcore_axis_name="s")
    fn = pl.kernel(
        _sc_body,
        out_type=jax.ShapeDtypeStruct((BH, POUT), f32),
        mesh=mesh,
        scratch_types=(
            [pltpu.VMEM((CH, POUT), f32) for _ in range(12)]
            + [pltpu.VMEM((CH, POUT), f32), pltpu.VMEM((CH, POUT), f32)]
            + [pltpu.SemaphoreType.DMA] * 4
        ),
        compiler_params=pltpu.CompilerParams(
            needs_layout_passes=False,
            use_tc_tiling_on_sc=False))
    return fn(pin)


def _final_body(p0, p1, f1W, f1b, f2w, f2b,
                outT, x1_ref, ti, tw, ai, aw, vi, vw, si, sw):
    half = pl.program_id(0) >= (B // 2) // p0.shape[0]
    pv = jnp.where(half, p1[...], p0[...])
    feat = pv[:, 0:24]
    x1 = jnp.dot(feat, f1W[...], preferred_element_type=jnp.float32) + f1b[...]
    x1_ref[...] = x1
    xr = jnp.maximum(x1, 0.0)
    ov = jnp.sum(xr * f2w[...], axis=1, keepdims=True) + f2b[...]
    outT[...] = jnp.transpose(ov)
    pT = jnp.transpose(pv)
    ti[...] = lax.bitcast_convert_type(pT[I_T:I_T + 7, :], jnp.int32)
    ai[...] = lax.bitcast_convert_type(pT[I_A:I_A + 7, :], jnp.int32)
    vi[...] = lax.bitcast_convert_type(pT[I_V:I_V + 7, :], jnp.int32)
    si[...] = lax.bitcast_convert_type(pT[I_S:I_S + 2, :], jnp.int32)
    tw[...] = pT[W_T:W_T + 7, :]
    aw[...] = pT[W_A:W_A + 7, :]
    vw[...] = pT[W_V:W_V + 7, :]
    sw[...] = pT[W_S:W_S + 2, :]


def _tc_final(pout0, pout1, f1Wp, f1b, f2w, f2b, R3=2048):
    grid = (B // R3,)
    nh = (B // 2) // R3
    f32, i32 = jnp.float32, jnp.int32

    def rows_spec(cols):
        return pl.BlockSpec((R3, cols), lambda i: (i, 0))

    def half_spec(h):
        if h == 0:
            return pl.BlockSpec((R3, POUT), lambda i: (jnp.minimum(i, nh - 1), 0))
        return pl.BlockSpec((R3, POUT),
                            lambda i: (jnp.maximum(i - nh, 0), 0))

    def colsT_spec(rows):
        return pl.BlockSpec((rows, R3), lambda i: (0, i))

    def full_spec(arr):
        return pl.BlockSpec(arr.shape, lambda i: (0,) * arr.ndim)

    return pl.pallas_call(
        _final_body, grid=grid,
        in_specs=[half_spec(0), half_spec(1), full_spec(f1Wp),
                  full_spec(f1b), full_spec(f2w), full_spec(f2b)],
        out_specs=[colsT_spec(1), rows_spec(256),
                   colsT_spec(7), colsT_spec(7), colsT_spec(7), colsT_spec(7),
                   colsT_spec(7), colsT_spec(7), colsT_spec(2), colsT_spec(2)],
        out_shape=[jax.ShapeDtypeStruct((1, B), f32),
                   jax.ShapeDtypeStruct((B, 256), f32),
                   jax.ShapeDtypeStruct((7, B), i32),
                   jax.ShapeDtypeStruct((7, B), f32),
                   jax.ShapeDtypeStruct((7, B), i32),
                   jax.ShapeDtypeStruct((7, B), f32),
                   jax.ShapeDtypeStruct((7, B), i32),
                   jax.ShapeDtypeStruct((7, B), f32),
                   jax.ShapeDtypeStruct((2, B), i32),
                   jax.ShapeDtypeStruct((2, B), f32)],
    )(pout0, pout1, f1Wp, f1b, f2w, f2b)


def kernel(thick, area, vol, sub_vol,
           tW1, tb1, tW2, tb2, aW1, ab1, aW2, ab2,
           vW1, vb1, vW2, vb2, sW1, sb1, sW2, sb2,
           f1W, f1b, f2W, f2b):
    f32 = jnp.float32
    pad60 = jnp.full((60,), NEG, f32)

    def padw_t(W1, b1, W2, b2):
        W2p = jnp.pad(W2, ((0, 0), (0, 60)))
        b2p = jnp.concatenate([b2, pad60])
        return (W1.T, b1.reshape(-1, 1), W2p.T, b2p.reshape(-1, 1))

    tw = padw_t(tW1, tb1, tW2, tb2)
    aw = padw_t(aW1, ab1, aW2, ab2)
    vw = padw_t(vW1, vb1, vW2, vb2)
    sw = (sW1.T, sb1.reshape(-1, 1), sW2.T, sb2.reshape(-1, 1))

    pin0 = _tc_scores(thick.T, area.T, vol.T, sub_vol.T, tw, aw, vw, sw, 0)
    pout0 = _sc_topk(pin0)
    pin1 = _tc_scores(thick.T, area.T, vol.T, sub_vol.T, tw, aw, vw, sw, 1)
    pout1 = _sc_topk(pin1)

    f1Wp = jnp.pad(f1W, ((0, 1), (0, 0)))
    (outT, x1, tiT, twT, aiT, awT, viT, vwT, siT, swT) = _tc_final(
        pout0, pout1, f1Wp, f1b.reshape(1, 256), f2W.reshape(1, 256),
        f2b.reshape(1, 1))

    return (outT.T, x1, tiT.T, twT.T, aiT.T, awT.T, viT.T, vwT.T,
            siT.T, swT.T)


# speedup vs baseline: 1.1402x; 1.1402x over previous
import functools

import jax
import jax.numpy as jnp
from jax import lax
from jax.experimental import pallas as pl
from jax.experimental.pallas import tpu as pltpu
from jax.experimental.pallas import tpu_sc as plsc

B = 16384
NEG = -1e30
NC = 2
NS = 16
NW = NC * NS
RPW = B // NW
CH = 64
NCH = RPW // CH

NSEC = 6
RB = 1024
SECROWS = NSEC * RB

POUT = 128
F_T, F_A, F_V, F_S = 0, 7, 14, 21
I_T, I_A, I_V, I_S = 24, 31, 38, 45
W_T, W_A, W_V, W_S = 47, 54, 61, 68


def _scores_body(tT, aT, vT, sT,
                 tW1, tb1, tW2, tb2,
                 aW1, ab1, aW2, ab2,
                 vW1, vb1, vW2, vb2,
                 sW1, sb1, sW2, sb2,
                 o):
    def mlp_t(xT, W1t, b1c, W2t, b2c):
        h = jnp.maximum(
            jnp.dot(W1t[...], xT, preferred_element_type=jnp.float32)
            + b1c[...], 0.0)
        return jnp.dot(W2t[...], h, preferred_element_type=jnp.float32) + b2c[...]

    tv, av, vv, sv = tT[...], aT[...], vT[...], sT[...]
    R = RB
    o[0:R, :] = jnp.transpose(mlp_t(tv, tW1, tb1, tW2, tb2))
    o[R:2 * R, :] = jnp.transpose(mlp_t(av, aW1, ab1, aW2, ab2))
    o[2 * R:3 * R, :] = jnp.transpose(mlp_t(vv, vW1, vb1, vW2, vb2))
    o[3 * R:4 * R, 0:16] = jnp.transpose(mlp_t(sv, sW1, sb1, sW2, sb2))
    o[3 * R:4 * R, 16:84] = jnp.transpose(tv)
    o[3 * R:4 * R, 84:100] = jnp.transpose(sv)
    o[4 * R:5 * R, 0:68] = jnp.transpose(av)
    o[5 * R:6 * R, 0:68] = jnp.transpose(vv)


def _tc_scores(tT, aT, vT, sT, tw, aw, vw, sw):
    grid = (B // RB,)

    def dataT_spec(rows):
        return pl.BlockSpec((rows, RB), lambda i: (0, i))

    def full_spec(arr):
        return pl.BlockSpec(arr.shape, lambda i: (0,) * arr.ndim)

    in_specs = [dataT_spec(68), dataT_spec(68), dataT_spec(68), dataT_spec(16)]
    ws = list(tw) + list(aw) + list(vw) + list(sw)
    in_specs += [full_spec(w) for w in ws]
    return pl.pallas_call(
        _scores_body, grid=grid, in_specs=in_specs,
        out_specs=pl.BlockSpec((SECROWS, POUT), lambda i: (i, 0)),
        out_shape=jax.ShapeDtypeStruct((B // RB * SECROWS, POUT), jnp.float32),
    )(tT, aT, vT, sT, *ws)


def _merge(ka, va, kb, vb):
    rkb = jnp.flip(kb)
    rvb = jnp.flip(vb)
    c = ka >= rkb
    hk = jnp.where(c, ka, rkb)
    hv = jnp.where(c, va, rvb)
    return plsc.sort_key_val(hk, hv, descending=True)


def _topk_row(sc_ref, r, sc_off, nchunk, lane):
    ks, vs = [], []
    for j in range(nchunk):
        key = sc_ref[r, pl.ds(sc_off + j * 16, 16)]
        kk, vv = plsc.sort_key_val(key, lane + j * 16, descending=True)
        ks.append(kk)
        vs.append(vv)
    while len(ks) > 1:
        nk, nv = [], []
        for i in range(0, len(ks) - 1, 2):
            kk, vv = _merge(ks[i], vs[i], ks[i + 1], vs[i + 1])
            nk.append(kk)
            nv.append(vv)
        if len(ks) % 2:
            nk.append(ks[-1])
            nv.append(vs[-1])
        ks, vs = nk, nv
    return ks[0], vs[0]


def _branch_row(sc_ref, sc_off, x_ref, x_off, po, r, rvec, nchunk, mk, lane,
                f_off, i_off, w_off, fm):
    keys, vals = _topk_row(sc_ref, r, sc_off, nchunk, lane)
    mx = jnp.max(keys)
    e = jnp.where(mk, jnp.exp(keys - mx), 0.0)
    w = e / jnp.sum(e)
    idx = jnp.where(mk, vals, 0)
    xs = plsc.load_gather(x_ref, [rvec, x_off + idx], mask=mk)
    wt = jnp.where(mk, xs * w, 0.0)
    plsc.store_scatter(po, [rvec, lane + i_off],
                       plsc.bitcast(idx, jnp.float32), mask=mk)
    plsc.store_scatter(po, [rvec, lane + w_off], w, mask=mk)
    plsc.store_scatter(po, [rvec, lane + f_off], wt, mask=fm)


def _sc_body(pin_h, pout_h,
             ct0, ca0, cv0, xd0, xe0, xf0,
             ct1, ca1, cv1, xd1, xe1, xf1,
             po0, po1, si0, si1, so0, so1):
    wid = lax.axis_index("s") * NC + lax.axis_index("c")
    lane = lax.iota(jnp.int32, 16)
    m7 = lane < 7
    m2 = lane < 2
    m3 = lane < 3
    secbase = (wid // 2) * SECROWS + (wid % 2) * RPW

    bufs = ((ct0, ca0, cv0, xd0, xe0, xf0),
            (ct1, ca1, cv1, xd1, xe1, xf1))
    pos = (po0, po1)
    sis = (si0, si1)
    sos = (so0, so1)

    def start_in(c, bsel):
        sem = sis[bsel]
        for s, dst in enumerate(bufs[bsel]):
            pltpu.async_copy(
                pin_h.at[pl.ds(secbase + s * RB + c * CH, CH)], dst, sem)

    def wait_in(bsel):
        for dst in bufs[bsel]:
            pltpu.make_async_copy(pin_h.at[pl.ds(0, CH)], dst,
                                  sis[bsel]).wait()

    def start_out(c, bsel):
        pltpu.async_copy(
            pos[bsel], pout_h.at[pl.ds(wid * RPW + c * CH, CH)], sos[bsel])

    def wait_out(bsel):
        pltpu.make_async_copy(pout_h.at[pl.ds(0, CH)], pos[bsel],
                              sos[bsel]).wait()

    def compute_chunk(bsel):
        sct, sca, scv, xd, xe, xf = bufs[bsel]
        po = pos[bsel]

        @plsc.parallel_loop(0, CH, unroll=2)
        def _(r):
            rvec = jnp.full((16,), r, jnp.int32)
            _branch_row(sct, 0, xd, 16, po, r, rvec, 5, m7, lane,
                        F_T, I_T, W_T, m7)
            _branch_row(sca, 0, xe, 0, po, r, rvec, 5, m7, lane,
                        F_A, I_A, W_A, m7)
            _branch_row(scv, 0, xf, 0, po, r, rvec, 5, m7, lane,
                        F_V, I_V, W_V, m7)
            _branch_row(xd, 0, xd, 84, po, r, rvec, 1, m2, lane,
                        F_S, I_S, W_S, m3)

    start_in(0, 0)

    def pair_body(c2, _):
        c = 2 * c2
        start_in(c + 1, 1)
        wait_in(0)

        @pl.when(c2 > 0)
        def _():
            wait_out(0)

        compute_chunk(0)
        start_out(c, 0)

        @pl.when(c2 < NCH // 2 - 1)
        def _():
            start_in(c + 2, 0)

        wait_in(1)

        @pl.when(c2 > 0)
        def _():
            wait_out(1)

        compute_chunk(1)
        start_out(c + 1, 1)
        return 0

    lax.fori_loop(0, NCH // 2, pair_body, 0)
    wait_out(0)
    wait_out(1)


def _sc_topk(pin):
    f32 = jnp.float32
    mesh = plsc.VectorSubcoreMesh(core_axis_name="c", subcore_axis_name="s")
    fn = pl.kernel(
        _sc_body,
        out_type=jax.ShapeDtypeStruct((B, POUT), f32),
        mesh=mesh,
        scratch_types=(
            [pltpu.VMEM((CH, POUT), f32) for _ in range(12)]
            + [pltpu.VMEM((CH, POUT), f32), pltpu.VMEM((CH, POUT), f32)]
            + [pltpu.SemaphoreType.DMA] * 4
        ),
        compiler_params=pltpu.CompilerParams(
            needs_layout_passes=False,
            use_tc_tiling_on_sc=False))
    return fn(pin)


def _final_body(p, f1W, f1b, f2w, f2b,
                outT, x1_ref, ti, tw, ai, aw, vi, vw, si, sw):
    pv = p[...]
    feat = pv[:, 0:24]
    x1 = jnp.dot(feat, f1W[...], preferred_element_type=jnp.float32) + f1b[...]
    x1_ref[...] = x1
    xr = jnp.maximum(x1, 0.0)
    ov = jnp.sum(xr * f2w[...], axis=1, keepdims=True) + f2b[...]
    outT[...] = jnp.transpose(ov)
    pT = jnp.transpose(pv)
    ti[...] = lax.bitcast_convert_type(pT[I_T:I_T + 7, :], jnp.int32)
    ai[...] = lax.bitcast_convert_type(pT[I_A:I_A + 7, :], jnp.int32)
    vi[...] = lax.bitcast_convert_type(pT[I_V:I_V + 7, :], jnp.int32)
    si[...] = lax.bitcast_convert_type(pT[I_S:I_S + 2, :], jnp.int32)
    tw[...] = pT[W_T:W_T + 7, :]
    aw[...] = pT[W_A:W_A + 7, :]
    vw[...] = pT[W_V:W_V + 7, :]
    sw[...] = pT[W_S:W_S + 2, :]


def _tc_final(pout, f1Wp, f1b, f2w, f2b, R3=2048):
    grid = (B // R3,)
    f32, i32 = jnp.float32, jnp.int32

    def rows_spec(cols):
        return pl.BlockSpec((R3, cols), lambda i: (i, 0))

    def colsT_spec(rows):
        return pl.BlockSpec((rows, R3), lambda i: (0, i))

    def full_spec(arr):
        return pl.BlockSpec(arr.shape, lambda i: (0,) * arr.ndim)

    return pl.pallas_call(
        _final_body, grid=grid,
        in_specs=[rows_spec(POUT), full_spec(f1Wp), full_spec(f1b),
                  full_spec(f2w), full_spec(f2b)],
        out_specs=[colsT_spec(1), rows_spec(256),
                   colsT_spec(7), colsT_spec(7), colsT_spec(7), colsT_spec(7),
                   colsT_spec(7), colsT_spec(7), colsT_spec(2), colsT_spec(2)],
        out_shape=[jax.ShapeDtypeStruct((1, B), f32),
                   jax.ShapeDtypeStruct((B, 256), f32),
                   jax.ShapeDtypeStruct((7, B), i32),
                   jax.ShapeDtypeStruct((7, B), f32),
                   jax.ShapeDtypeStruct((7, B), i32),
                   jax.ShapeDtypeStruct((7, B), f32),
                   jax.ShapeDtypeStruct((7, B), i32),
                   jax.ShapeDtypeStruct((7, B), f32),
                   jax.ShapeDtypeStruct((2, B), i32),
                   jax.ShapeDtypeStruct((2, B), f32)],
    )(pout, f1Wp, f1b, f2w, f2b)


def kernel(thick, area, vol, sub_vol,
           tW1, tb1, tW2, tb2, aW1, ab1, aW2, ab2,
           vW1, vb1, vW2, vb2, sW1, sb1, sW2, sb2,
           f1W, f1b, f2W, f2b):
    f32 = jnp.float32
    pad60 = jnp.full((60,), NEG, f32)

    def padw_t(W1, b1, W2, b2):
        W2p = jnp.pad(W2, ((0, 0), (0, 60)))
        b2p = jnp.concatenate([b2, pad60])
        return (W1.T, b1.reshape(-1, 1), W2p.T, b2p.reshape(-1, 1))

    tw = padw_t(tW1, tb1, tW2, tb2)
    aw = padw_t(aW1, ab1, aW2, ab2)
    vw = padw_t(vW1, vb1, vW2, vb2)
    sw = (sW1.T, sb1.reshape(-1, 1), sW2.T, sb2.reshape(-1, 1))

    pin = _tc_scores(thick.T, area.T, vol.T, sub_vol.T, tw, aw, vw, sw)
    pout = _sc_topk(pin)

    f1Wp = jnp.pad(f1W, ((0, 1), (0, 0)))
    (outT, x1, tiT, twT, aiT, awT, viT, vwT, siT, swT) = _tc_final(
        pout, f1Wp, f1b.reshape(1, 256), f2W.reshape(1, 256),
        f2b.reshape(1, 1))

    return (outT.T, x1, tiT.T, twT.T, aiT.T, awT.T, viT.T, vwT.T,
            siT.T, swT.T)


# speedup vs baseline: 1.1879x; 1.0418x over previous
import functools

import jax
import jax.numpy as jnp
from jax import lax
from jax.experimental import pallas as pl
from jax.experimental.pallas import tpu as pltpu
from jax.experimental.pallas import tpu_sc as plsc

B = 16384
NEG = -1e30
NC = 2
NS = 16
NW = NC * NS
RPW = B // NW
CH = 64
NCH = RPW // CH

NSEC = 6
RB = 2048
SECROWS = NSEC * RB

POUT = 128
F_T, F_A, F_V, F_S = 0, 7, 14, 21
I_T, I_A, I_V, I_S = 24, 31, 38, 45
W_T, W_A, W_V, W_S = 47, 54, 61, 68


def _scores_body(tT, aT, vT, sT,
                 tW1, tb1, tW2, tb2,
                 aW1, ab1, aW2, ab2,
                 vW1, vb1, vW2, vb2,
                 sW1, sb1, sW2, sb2,
                 o):
    def mlp_t(xT, W1t, b1c, W2t, b2c):
        h = jnp.maximum(
            jnp.dot(W1t[...], xT, preferred_element_type=jnp.float32)
            + b1c[...], 0.0)
        return jnp.dot(W2t[...], h, preferred_element_type=jnp.float32) + b2c[...]

    tv, av, vv, sv = tT[...], aT[...], vT[...], sT[...]
    R = RB
    o[0:R, :] = jnp.transpose(mlp_t(tv, tW1, tb1, tW2, tb2))
    o[R:2 * R, :] = jnp.transpose(mlp_t(av, aW1, ab1, aW2, ab2))
    o[2 * R:3 * R, :] = jnp.transpose(mlp_t(vv, vW1, vb1, vW2, vb2))
    o[3 * R:4 * R, 0:16] = jnp.transpose(mlp_t(sv, sW1, sb1, sW2, sb2))
    o[3 * R:4 * R, 16:84] = jnp.transpose(tv)
    o[3 * R:4 * R, 84:100] = jnp.transpose(sv)
    o[4 * R:5 * R, 0:68] = jnp.transpose(av)
    o[5 * R:6 * R, 0:68] = jnp.transpose(vv)


def _tc_scores(tT, aT, vT, sT, tw, aw, vw, sw):
    grid = (B // RB,)

    def dataT_spec(rows):
        return pl.BlockSpec((rows, RB), lambda i: (0, i))

    def full_spec(arr):
        return pl.BlockSpec(arr.shape, lambda i: (0,) * arr.ndim)

    in_specs = [dataT_spec(68), dataT_spec(68), dataT_spec(68), dataT_spec(16)]
    ws = list(tw) + list(aw) + list(vw) + list(sw)
    in_specs += [full_spec(w) for w in ws]
    return pl.pallas_call(
        _scores_body, grid=grid, in_specs=in_specs,
        out_specs=pl.BlockSpec((SECROWS, POUT), lambda i: (i, 0)),
        out_shape=jax.ShapeDtypeStruct((B // RB * SECROWS, POUT), jnp.float32),
    )(tT, aT, vT, sT, *ws)


def _merge(ka, va, kb, vb):
    rkb = jnp.flip(kb)
    rvb = jnp.flip(vb)
    c = ka >= rkb
    hk = jnp.where(c, ka, rkb)
    hv = jnp.where(c, va, rvb)
    return plsc.sort_key_val(hk, hv, descending=True)


def _topk_row(sc_ref, r, sc_off, nchunk, lane):
    ks, vs = [], []
    for j in range(nchunk):
        key = sc_ref[r, pl.ds(sc_off + j * 16, 16)]
        kk, vv = plsc.sort_key_val(key, lane + j * 16, descending=True)
        ks.append(kk)
        vs.append(vv)
    while len(ks) > 1:
        nk, nv = [], []
        for i in range(0, len(ks) - 1, 2):
            kk, vv = _merge(ks[i], vs[i], ks[i + 1], vs[i + 1])
            nk.append(kk)
            nv.append(vv)
        if len(ks) % 2:
            nk.append(ks[-1])
            nv.append(vs[-1])
        ks, vs = nk, nv
    return ks[0], vs[0]


def _branch_row(sc_ref, sc_off, x_ref, x_off, po, r, rvec, nchunk, mk, lane,
                f_off, i_off, w_off, fm):
    keys, vals = _topk_row(sc_ref, r, sc_off, nchunk, lane)
    mx = jnp.max(keys)
    e = jnp.where(mk, jnp.exp(keys - mx), 0.0)
    w = e / jnp.sum(e)
    idx = jnp.where(mk, vals, 0)
    xs = plsc.load_gather(x_ref, [rvec, x_off + idx], mask=mk)
    wt = jnp.where(mk, xs * w, 0.0)
    plsc.store_scatter(po, [rvec, lane + i_off],
                       plsc.bitcast(idx, jnp.float32), mask=mk)
    plsc.store_scatter(po, [rvec, lane + w_off], w, mask=mk)
    plsc.store_scatter(po, [rvec, lane + f_off], wt, mask=fm)


def _sc_body(pin_h, pout_h,
             ct0, ca0, cv0, xd0, xe0, xf0,
             ct1, ca1, cv1, xd1, xe1, xf1,
             po0, po1, si0, si1, so0, so1):
    wid = lax.axis_index("s") * NC + lax.axis_index("c")
    lane = lax.iota(jnp.int32, 16)
    m7 = lane < 7
    m2 = lane < 2
    m3 = lane < 3
    secbase = (wid // 4) * SECROWS + (wid % 4) * RPW

    bufs = ((ct0, ca0, cv0, xd0, xe0, xf0),
            (ct1, ca1, cv1, xd1, xe1, xf1))
    pos = (po0, po1)
    sis = (si0, si1)
    sos = (so0, so1)

    def start_in(c, bsel):
        sem = sis[bsel]
        for s, dst in enumerate(bufs[bsel]):
            pltpu.async_copy(
                pin_h.at[pl.ds(secbase + s * RB + c * CH, CH)], dst, sem)

    def wait_in(bsel):
        for dst in bufs[bsel]:
            pltpu.make_async_copy(pin_h.at[pl.ds(0, CH)], dst,
                                  sis[bsel]).wait()

    def start_out(c, bsel):
        pltpu.async_copy(
            pos[bsel], pout_h.at[pl.ds(wid * RPW + c * CH, CH)], sos[bsel])

    def wait_out(bsel):
        pltpu.make_async_copy(pout_h.at[pl.ds(0, CH)], pos[bsel],
                              sos[bsel]).wait()

    def compute_chunk(bsel):
        sct, sca, scv, xd, xe, xf = bufs[bsel]
        po = pos[bsel]

        @plsc.parallel_loop(0, CH, unroll=2)
        def _(r):
            rvec = jnp.full((16,), r, jnp.int32)
            _branch_row(sct, 0, xd, 16, po, r, rvec, 5, m7, lane,
                        F_T, I_T, W_T, m7)
            _branch_row(sca, 0, xe, 0, po, r, rvec, 5, m7, lane,
                        F_A, I_A, W_A, m7)
            _branch_row(scv, 0, xf, 0, po, r, rvec, 5, m7, lane,
                        F_V, I_V, W_V, m7)
            _branch_row(xd, 0, xd, 84, po, r, rvec, 1, m2, lane,
                        F_S, I_S, W_S, m3)

    start_in(0, 0)

    def pair_body(c2, _):
        c = 2 * c2
        start_in(c + 1, 1)
        wait_in(0)

        @pl.when(c2 > 0)
        def _():
            wait_out(0)

        compute_chunk(0)
        start_out(c, 0)

        @pl.when(c2 < NCH // 2 - 1)
        def _():
            start_in(c + 2, 0)

        wait_in(1)

        @pl.when(c2 > 0)
        def _():
            wait_out(1)

        compute_chunk(1)
        start_out(c + 1, 1)
        return 0

    lax.fori_loop(0, NCH // 2, pair_body, 0)
    wait_out(0)
    wait_out(1)


def _sc_topk(pin):
    f32 = jnp.float32
    mesh = plsc.VectorSubcoreMesh(core_axis_name="c", subcore_axis_name="s")
    fn = pl.kernel(
        _sc_body,
        out_type=jax.ShapeDtypeStruct((B, POUT), f32),
        mesh=mesh,
        scratch_types=(
            [pltpu.VMEM((CH, POUT), f32) for _ in range(12)]
            + [pltpu.VMEM((CH, POUT), f32), pltpu.VMEM((CH, POUT), f32)]
            + [pltpu.SemaphoreType.DMA] * 4
        ),
        compiler_params=pltpu.CompilerParams(
            needs_layout_passes=False,
            use_tc_tiling_on_sc=False))
    return fn(pin)


def _final_body(p, f1W, f1b, f2w, f2b,
                outT, x1_ref, ti, tw, ai, aw, vi, vw, si, sw):
    pv = p[...]
    feat = pv[:, 0:24]
    x1 = jnp.dot(feat, f1W[...], preferred_element_type=jnp.float32) + f1b[...]
    x1_ref[...] = x1
    xr = jnp.maximum(x1, 0.0)
    ov = jnp.sum(xr * f2w[...], axis=1, keepdims=True) + f2b[...]
    outT[...] = jnp.transpose(ov)
    pT = jnp.transpose(pv)
    ti[...] = lax.bitcast_convert_type(pT[I_T:I_T + 7, :], jnp.int32)
    ai[...] = lax.bitcast_convert_type(pT[I_A:I_A + 7, :], jnp.int32)
    vi[...] = lax.bitcast_convert_type(pT[I_V:I_V + 7, :], jnp.int32)
    si[...] = lax.bitcast_convert_type(pT[I_S:I_S + 2, :], jnp.int32)
    tw[...] = pT[W_T:W_T + 7, :]
    aw[...] = pT[W_A:W_A + 7, :]
    vw[...] = pT[W_V:W_V + 7, :]
    sw[...] = pT[W_S:W_S + 2, :]


def _tc_final(pout, f1Wp, f1b, f2w, f2b, R3=2048):
    grid = (B // R3,)
    f32, i32 = jnp.float32, jnp.int32

    def rows_spec(cols):
        return pl.BlockSpec((R3, cols), lambda i: (i, 0))

    def colsT_spec(rows):
        return pl.BlockSpec((rows, R3), lambda i: (0, i))

    def full_spec(arr):
        return pl.BlockSpec(arr.shape, lambda i: (0,) * arr.ndim)

    return pl.pallas_call(
        _final_body, grid=grid,
        in_specs=[rows_spec(POUT), full_spec(f1Wp), full_spec(f1b),
                  full_spec(f2w), full_spec(f2b)],
        out_specs=[colsT_spec(1), rows_spec(256),
                   colsT_spec(7), colsT_spec(7), colsT_spec(7), colsT_spec(7),
                   colsT_spec(7), colsT_spec(7), colsT_spec(2), colsT_spec(2)],
        out_shape=[jax.ShapeDtypeStruct((1, B), f32),
                   jax.ShapeDtypeStruct((B, 256), f32),
                   jax.ShapeDtypeStruct((7, B), i32),
                   jax.ShapeDtypeStruct((7, B), f32),
                   jax.ShapeDtypeStruct((7, B), i32),
                   jax.ShapeDtypeStruct((7, B), f32),
                   jax.ShapeDtypeStruct((7, B), i32),
                   jax.ShapeDtypeStruct((7, B), f32),
                   jax.ShapeDtypeStruct((2, B), i32),
                   jax.ShapeDtypeStruct((2, B), f32)],
    )(pout, f1Wp, f1b, f2w, f2b)


def kernel(thick, area, vol, sub_vol,
           tW1, tb1, tW2, tb2, aW1, ab1, aW2, ab2,
           vW1, vb1, vW2, vb2, sW1, sb1, sW2, sb2,
           f1W, f1b, f2W, f2b):
    f32 = jnp.float32
    pad60 = jnp.full((60,), NEG, f32)

    def padw_t(W1, b1, W2, b2):
        W2p = jnp.pad(W2, ((0, 0), (0, 60)))
        b2p = jnp.concatenate([b2, pad60])
        return (W1.T, b1.reshape(-1, 1), W2p.T, b2p.reshape(-1, 1))

    tw = padw_t(tW1, tb1, tW2, tb2)
    aw = padw_t(aW1, ab1, aW2, ab2)
    vw = padw_t(vW1, vb1, vW2, vb2)
    sw = (sW1.T, sb1.reshape(-1, 1), sW2.T, sb2.reshape(-1, 1))

    pin = _tc_scores(thick.T, area.T, vol.T, sub_vol.T, tw, aw, vw, sw)
    pout = _sc_topk(pin)

    f1Wp = jnp.pad(f1W, ((0, 1), (0, 0)))
    (outT, x1, tiT, twT, aiT, awT, viT, vwT, siT, swT) = _tc_final(
        pout, f1Wp, f1b.reshape(1, 256), f2W.reshape(1, 256),
        f2b.reshape(1, 1))

    return (outT.T, x1, tiT.T, twT.T, aiT.T, awT.T, viT.T, vwT.T,
            siT.T, swT.T)


# speedup vs baseline: 1.1922x; 1.0036x over previous
import functools

import jax
import jax.numpy as jnp
from jax import lax
from jax.experimental import pallas as pl
from jax.experimental.pallas import tpu as pltpu
from jax.experimental.pallas import tpu_sc as plsc

B = 16384
NEG = -1e30
NC = 2
NS = 16
NW = NC * NS
RPW = B // NW
CH = 64
NCH = RPW // CH

NSEC = 6
RB = 4096
SECROWS = NSEC * RB

POUT = 128
F_T, F_A, F_V, F_S = 0, 7, 14, 21
I_T, I_A, I_V, I_S = 24, 31, 38, 45
W_T, W_A, W_V, W_S = 47, 54, 61, 68


def _scores_body(tT, aT, vT, sT,
                 tW1, tb1, tW2, tb2,
                 aW1, ab1, aW2, ab2,
                 vW1, vb1, vW2, vb2,
                 sW1, sb1, sW2, sb2,
                 o):
    def mlp_t(xT, W1t, b1c, W2t, b2c):
        h = jnp.maximum(
            jnp.dot(W1t[...], xT, preferred_element_type=jnp.float32)
            + b1c[...], 0.0)
        return jnp.dot(W2t[...], h, preferred_element_type=jnp.float32) + b2c[...]

    tv, av, vv, sv = tT[...], aT[...], vT[...], sT[...]
    R = RB
    o[0:R, :] = jnp.transpose(mlp_t(tv, tW1, tb1, tW2, tb2))
    o[R:2 * R, :] = jnp.transpose(mlp_t(av, aW1, ab1, aW2, ab2))
    o[2 * R:3 * R, :] = jnp.transpose(mlp_t(vv, vW1, vb1, vW2, vb2))
    o[3 * R:4 * R, 0:16] = jnp.transpose(mlp_t(sv, sW1, sb1, sW2, sb2))
    o[3 * R:4 * R, 16:84] = jnp.transpose(tv)
    o[3 * R:4 * R, 84:100] = jnp.transpose(sv)
    o[4 * R:5 * R, 0:68] = jnp.transpose(av)
    o[5 * R:6 * R, 0:68] = jnp.transpose(vv)


def _tc_scores(tT, aT, vT, sT, tw, aw, vw, sw):
    grid = (B // RB,)

    def dataT_spec(rows):
        return pl.BlockSpec((rows, RB), lambda i: (0, i))

    def full_spec(arr):
        return pl.BlockSpec(arr.shape, lambda i: (0,) * arr.ndim)

    in_specs = [dataT_spec(68), dataT_spec(68), dataT_spec(68), dataT_spec(16)]
    ws = list(tw) + list(aw) + list(vw) + list(sw)
    in_specs += [full_spec(w) for w in ws]
    return pl.pallas_call(
        _scores_body, grid=grid, in_specs=in_specs,
        out_specs=pl.BlockSpec((SECROWS, POUT), lambda i: (i, 0)),
        out_shape=jax.ShapeDtypeStruct((B // RB * SECROWS, POUT), jnp.float32),
    )(tT, aT, vT, sT, *ws)


def _merge(ka, va, kb, vb):
    rkb = jnp.flip(kb)
    rvb = jnp.flip(vb)
    c = ka >= rkb
    hk = jnp.where(c, ka, rkb)
    hv = jnp.where(c, va, rvb)
    return plsc.sort_key_val(hk, hv, descending=True)


def _topk_row(sc_ref, r, sc_off, nchunk, lane):
    ks, vs = [], []
    for j in range(nchunk):
        key = sc_ref[r, pl.ds(sc_off + j * 16, 16)]
        kk, vv = plsc.sort_key_val(key, lane + j * 16, descending=True)
        ks.append(kk)
        vs.append(vv)
    while len(ks) > 1:
        nk, nv = [], []
        for i in range(0, len(ks) - 1, 2):
            kk, vv = _merge(ks[i], vs[i], ks[i + 1], vs[i + 1])
            nk.append(kk)
            nv.append(vv)
        if len(ks) % 2:
            nk.append(ks[-1])
            nv.append(vs[-1])
        ks, vs = nk, nv
    return ks[0], vs[0]


def _branch_row(sc_ref, sc_off, x_ref, x_off, po, r, rvec, nchunk, mk, lane,
                f_off, i_off, w_off, fm):
    keys, vals = _topk_row(sc_ref, r, sc_off, nchunk, lane)
    mx = jnp.max(keys)
    e = jnp.where(mk, jnp.exp(keys - mx), 0.0)
    w = e / jnp.sum(e)
    idx = jnp.where(mk, vals, 0)
    xs = plsc.load_gather(x_ref, [rvec, x_off + idx], mask=mk)
    wt = jnp.where(mk, xs * w, 0.0)
    plsc.store_scatter(po, [rvec, lane + i_off],
                       plsc.bitcast(idx, jnp.float32), mask=mk)
    plsc.store_scatter(po, [rvec, lane + w_off], w, mask=mk)
    plsc.store_scatter(po, [rvec, lane + f_off], wt, mask=fm)


def _sc_body(pin_h, pout_h,
             ct0, ca0, cv0, xd0, xe0, xf0,
             ct1, ca1, cv1, xd1, xe1, xf1,
             po0, po1, si0, si1, so0, so1):
    wid = lax.axis_index("s") * NC + lax.axis_index("c")
    lane = lax.iota(jnp.int32, 16)
    m7 = lane < 7
    m2 = lane < 2
    m3 = lane < 3
    secbase = (wid // 8) * SECROWS + (wid % 8) * RPW

    bufs = ((ct0, ca0, cv0, xd0, xe0, xf0),
            (ct1, ca1, cv1, xd1, xe1, xf1))
    pos = (po0, po1)
    sis = (si0, si1)
    sos = (so0, so1)

    def start_in(c, bsel):
        sem = sis[bsel]
        for s, dst in enumerate(bufs[bsel]):
            pltpu.async_copy(
                pin_h.at[pl.ds(secbase + s * RB + c * CH, CH)], dst, sem)

    def wait_in(bsel):
        for dst in bufs[bsel]:
            pltpu.make_async_copy(pin_h.at[pl.ds(0, CH)], dst,
                                  sis[bsel]).wait()

    def start_out(c, bsel):
        pltpu.async_copy(
            pos[bsel], pout_h.at[pl.ds(wid * RPW + c * CH, CH)], sos[bsel])

    def wait_out(bsel):
        pltpu.make_async_copy(pout_h.at[pl.ds(0, CH)], pos[bsel],
                              sos[bsel]).wait()

    def compute_chunk(bsel):
        sct, sca, scv, xd, xe, xf = bufs[bsel]
        po = pos[bsel]

        @plsc.parallel_loop(0, CH, unroll=2)
        def _(r):
            rvec = jnp.full((16,), r, jnp.int32)
            _branch_row(sct, 0, xd, 16, po, r, rvec, 5, m7, lane,
                        F_T, I_T, W_T, m7)
            _branch_row(sca, 0, xe, 0, po, r, rvec, 5, m7, lane,
                        F_A, I_A, W_A, m7)
            _branch_row(scv, 0, xf, 0, po, r, rvec, 5, m7, lane,
                        F_V, I_V, W_V, m7)
            _branch_row(xd, 0, xd, 84, po, r, rvec, 1, m2, lane,
                        F_S, I_S, W_S, m3)

    start_in(0, 0)

    def pair_body(c2, _):
        c = 2 * c2
        start_in(c + 1, 1)
        wait_in(0)

        @pl.when(c2 > 0)
        def _():
            wait_out(0)

        compute_chunk(0)
        start_out(c, 0)

        @pl.when(c2 < NCH // 2 - 1)
        def _():
            start_in(c + 2, 0)

        wait_in(1)

        @pl.when(c2 > 0)
        def _():
            wait_out(1)

        compute_chunk(1)
        start_out(c + 1, 1)
        return 0

    lax.fori_loop(0, NCH // 2, pair_body, 0)
    wait_out(0)
    wait_out(1)


def _sc_topk(pin):
    f32 = jnp.float32
    mesh = plsc.VectorSubcoreMesh(core_axis_name="c", subcore_axis_name="s")
    fn = pl.kernel(
        _sc_body,
        out_type=jax.ShapeDtypeStruct((B, POUT), f32),
        mesh=mesh,
        scratch_types=(
            [pltpu.VMEM((CH, POUT), f32) for _ in range(12)]
            + [pltpu.VMEM((CH, POUT), f32), pltpu.VMEM((CH, POUT), f32)]
            + [pltpu.SemaphoreType.DMA] * 4
        ),
        compiler_params=pltpu.CompilerParams(
            needs_layout_passes=False,
            use_tc_tiling_on_sc=False))
    return fn(pin)


def _final_body(p, f1W, f1b, f2w, f2b,
                outT, x1_ref, ti, tw, ai, aw, vi, vw, si, sw):
    pv = p[...]
    feat = pv[:, 0:24]
    x1 = jnp.dot(feat, f1W[...], preferred_element_type=jnp.float32) + f1b[...]
    x1_ref[...] = x1
    xr = jnp.maximum(x1, 0.0)
    ov = jnp.sum(xr * f2w[...], axis=1, keepdims=True) + f2b[...]
    outT[...] = jnp.transpose(ov)
    pT = jnp.transpose(pv)
    ti[...] = lax.bitcast_convert_type(pT[I_T:I_T + 7, :], jnp.int32)
    ai[...] = lax.bitcast_convert_type(pT[I_A:I_A + 7, :], jnp.int32)
    vi[...] = lax.bitcast_convert_type(pT[I_V:I_V + 7, :], jnp.int32)
    si[...] = lax.bitcast_convert_type(pT[I_S:I_S + 2, :], jnp.int32)
    tw[...] = pT[W_T:W_T + 7, :]
    aw[...] = pT[W_A:W_A + 7, :]
    vw[...] = pT[W_V:W_V + 7, :]
    sw[...] = pT[W_S:W_S + 2, :]


def _tc_final(pout, f1Wp, f1b, f2w, f2b, R3=2048):
    grid = (B // R3,)
    f32, i32 = jnp.float32, jnp.int32

    def rows_spec(cols):
        return pl.BlockSpec((R3, cols), lambda i: (i, 0))

    def colsT_spec(rows):
        return pl.BlockSpec((rows, R3), lambda i: (0, i))

    def full_spec(arr):
        return pl.BlockSpec(arr.shape, lambda i: (0,) * arr.ndim)

    return pl.pallas_call(
        _final_body, grid=grid,
        in_specs=[rows_spec(POUT), full_spec(f1Wp), full_spec(f1b),
                  full_spec(f2w), full_spec(f2b)],
        out_specs=[colsT_spec(1), rows_spec(256),
                   colsT_spec(7), colsT_spec(7), colsT_spec(7), colsT_spec(7),
                   colsT_spec(7), colsT_spec(7), colsT_spec(2), colsT_spec(2)],
        out_shape=[jax.ShapeDtypeStruct((1, B), f32),
                   jax.ShapeDtypeStruct((B, 256), f32),
                   jax.ShapeDtypeStruct((7, B), i32),
                   jax.ShapeDtypeStruct((7, B), f32),
                   jax.ShapeDtypeStruct((7, B), i32),
                   jax.ShapeDtypeStruct((7, B), f32),
                   jax.ShapeDtypeStruct((7, B), i32),
                   jax.ShapeDtypeStruct((7, B), f32),
                   jax.ShapeDtypeStruct((2, B), i32),
                   jax.ShapeDtypeStruct((2, B), f32)],
    )(pout, f1Wp, f1b, f2w, f2b)


def kernel(thick, area, vol, sub_vol,
           tW1, tb1, tW2, tb2, aW1, ab1, aW2, ab2,
           vW1, vb1, vW2, vb2, sW1, sb1, sW2, sb2,
           f1W, f1b, f2W, f2b):
    f32 = jnp.float32
    pad60 = jnp.full((60,), NEG, f32)

    def padw_t(W1, b1, W2, b2):
        W2p = jnp.pad(W2, ((0, 0), (0, 60)))
        b2p = jnp.concatenate([b2, pad60])
        return (W1.T, b1.reshape(-1, 1), W2p.T, b2p.reshape(-1, 1))

    tw = padw_t(tW1, tb1, tW2, tb2)
    aw = padw_t(aW1, ab1, aW2, ab2)
    vw = padw_t(vW1, vb1, vW2, vb2)
    sw = (sW1.T, sb1.reshape(-1, 1), sW2.T, sb2.reshape(-1, 1))

    pin = _tc_scores(thick.T, area.T, vol.T, sub_vol.T, tw, aw, vw, sw)
    pout = _sc_topk(pin)

    f1Wp = jnp.pad(f1W, ((0, 1), (0, 0)))
    (outT, x1, tiT, twT, aiT, awT, viT, vwT, siT, swT) = _tc_final(
        pout, f1Wp, f1b.reshape(1, 256), f2W.reshape(1, 256),
        f2b.reshape(1, 1))

    return (outT.T, x1, tiT.T, twT.T, aiT.T, awT.T, viT.T, vwT.T,
            siT.T, swT.T)
